# baseline (speedup 1.0000x reference)
import jax
import jax.numpy as jnp
from jax.experimental import pallas as pl
from jax.experimental.pallas import tpu as pltpu

N = 1024
H = 64
B = 32
R = 128
NBLK = N // R


def _gelu(x):
    return 0.5 * x * (1.0 + jax.lax.erf(x * 0.7071067811865476))


def _ggnn_kernel(rows_ref, cols_ref, nf_ref, w1_ref, b1_ref, w2_ref, b2_ref,
                 out_ref, ew_s, z1_s, d_s):
    i = pl.program_id(0)
    rows = rows_ref[:]
    cols = cols_ref[:]
    s = 0.5 * (rows + cols.T)
    sp = jnp.log1p(jnp.exp(s))
    rg = jax.lax.broadcasted_iota(jnp.int32, (R, N), 0) + i * R
    cg = jax.lax.broadcasted_iota(jnp.int32, (R, N), 1)
    ew = jnp.where(rg == cg, 1e-6, jnp.maximum(sp, 1e-6))
    ew_s[pl.ds(i * R, R), :] = ew

    db = jax.lax.rsqrt(jnp.sum(ew, axis=1, keepdims=True))
    d_s[pl.ds(i * R, R), :] = db

    nfb = nf_ref[:]
    x0 = jnp.sum(nfb[:, :B], axis=1, keepdims=True) * (1.0 / B)
    x1 = jnp.sum(nfb[:, B:], axis=1, keepdims=True) * (1.0 / B)
    xw1 = x0 * w1_ref[0:1, :] + x1 * w1_ref[1:2, :]

    contrib = jax.lax.dot_general(
        ew, db * xw1, (((0,), (0,)), ((), ())),
        preferred_element_type=jnp.float32)

    @pl.when(i == 0)
    def _():
        z1_s[:] = contrib

    @pl.when(i > 0)
    def _():
        z1_s[:] = z1_s[:] + contrib

    @pl.when(i == NBLK - 1)
    def _():
        d = d_s[:]
        h1 = _gelu(d * z1_s[:] + b1_ref[:])
        xw2 = jnp.dot(h1, w2_ref[:], preferred_element_type=jnp.float32)
        z2 = jnp.dot(ew_s[:], d * xw2, preferred_element_type=jnp.float32)
        out_ref[:] = _gelu(d * z2 + b2_ref[:])


def kernel(node_feats, X_for_graph, raw, W1, b1, W2, b2):
    del X_for_graph
    nf = jnp.transpose(node_feats, (1, 2, 0)).reshape(N, 2 * B)
    return pl.pallas_call(
        _ggnn_kernel,
        grid=(NBLK,),
        in_specs=[
            pl.BlockSpec((R, N), lambda i: (i, 0)),
            pl.BlockSpec((N, R), lambda i: (0, i)),
            pl.BlockSpec((R, 2 * B), lambda i: (i, 0)),
            pl.BlockSpec((2, H), lambda i: (0, 0)),
            pl.BlockSpec((1, H), lambda i: (0, 0)),
            pl.BlockSpec((H, H), lambda i: (0, 0)),
            pl.BlockSpec((1, H), lambda i: (0, 0)),
        ],
        out_specs=pl.BlockSpec((N, H), lambda i: (0, 0)),
        scratch_shapes=[
            pltpu.VMEM((N, N), jnp.float32),
            pltpu.VMEM((N, H), jnp.float32),
            pltpu.VMEM((N, 1), jnp.float32),
        ],
        out_shape=jax.ShapeDtypeStruct((N, H), jnp.float32),
    )(raw, raw, nf, W1, b1.reshape(1, H), W2, b2.reshape(1, H))
